# baseline (speedup 1.0000x reference)
import functools

import jax
import jax.numpy as jnp
from jax import lax
from jax.experimental import pallas as pl
from jax.experimental.pallas import tpu as pltpu
from jax.experimental.pallas import tpu_sc as plsc

N = 4096
C = 100000
L = 16
NS = 16
BW = N // NS
NCHUNK = BW // 128


def _sc_body(prob_hbm, tgt_hbm, rew_hbm, part_hbm, out_hbm,
             tgt_v, rew_v, idx_v, vals_v, acc_v, all_v, sem):
    sid = lax.axis_index("s")
    base = sid * BW

    pltpu.sync_copy(tgt_hbm.at[pl.ds(base, BW)], tgt_v)
    pltpu.sync_copy(rew_hbm.at[pl.ds(base, BW)], rew_v)

    lane = lax.iota(jnp.int32, L)
    for k in range(NCHUNK):
        for j in range(128 // L):
            rows = lane + (base + k * 128 + j * L)
            idx_v[k, pl.ds(j * L, L)] = rows * C + tgt_v[pl.ds(k * 128 + j * L, L)]

    copies = [
        pltpu.async_copy(prob_hbm.at[idx_v.at[k]], vals_v.at[k], sem)
        for k in range(NCHUNK)
    ]
    for cp in copies:
        cp.wait()

    acc = jnp.zeros((L,), jnp.float32)
    for k in range(NCHUNK):
        for j in range(128 // L):
            acc = acc + (vals_v[k, pl.ds(j * L, L)] *
                         rew_v[pl.ds(k * 128 + j * L, L)])
    acc_v[...] = acc

    pltpu.sync_copy(acc_v, part_hbm.at[sid])
    plsc.subcore_barrier()

    @pl.when(sid == 0)
    def _():
        pltpu.sync_copy(part_hbm, all_v)
        tot = jnp.zeros((L,), jnp.float32)
        for r in range(NS):
            tot = tot + all_v[r]
        s = tot[0]
        for i in range(1, L):
            s = s + tot[i]
        acc_v[...] = lax.broadcast_in_dim(-s, (L,), ())
        pltpu.sync_copy(acc_v, out_hbm)


@jax.jit
def _sc_loss(prob_flat, target, reward):
    mesh = plsc.VectorSubcoreMesh(
        core_axis_name="c", subcore_axis_name="s", num_cores=1)
    f = functools.partial(
        pl.kernel,
        out_type=(jax.ShapeDtypeStruct((NS, L), jnp.float32),
                  jax.ShapeDtypeStruct((L,), jnp.float32)),
        mesh=mesh,
        scratch_types=[
            pltpu.VMEM((BW,), jnp.int32),
            pltpu.VMEM((BW,), jnp.float32),
            pltpu.VMEM((NCHUNK, 128), jnp.int32),
            pltpu.VMEM((NCHUNK, 128), jnp.float32),
            pltpu.VMEM((L,), jnp.float32),
            pltpu.VMEM((NS, L), jnp.float32),
            pltpu.SemaphoreType.DMA,
        ],
    )(_sc_body)
    return f(prob_flat, target, reward)


def kernel(prob, target, reward):
    _, out = _sc_loss(prob.reshape(-1), target, reward)
    return out[0]


# speedup vs baseline: 2.3962x; 2.3962x over previous
import functools

import jax
import jax.numpy as jnp
from jax import lax
from jax.experimental import pallas as pl
from jax.experimental.pallas import tpu as pltpu
from jax.experimental.pallas import tpu_sc as plsc

N = 4096
C = 100000
L = 16
NC = 2
NS = 16
NW = NC * NS
BW = N // NW
CH = 32
NCHK = BW // CH
NG = BW // L


def _sc_body(prob_hbm, tgt_hbm, rew_hbm, part_hbm, out_hbm,
             tgt_v, rew_v, vals_v, acc_v, all_v, sem0, sem1):
    cid = lax.axis_index("c")
    sid = lax.axis_index("s")
    wid = cid * NS + sid
    base = wid * BW

    pltpu.sync_copy(tgt_hbm.at[pl.ds(base, BW)], tgt_v)
    pltpu.sync_copy(rew_hbm.at[pl.ds(base, BW)], rew_v)

    lane = lax.iota(jnp.int32, L)
    sems = [sem0, sem1]

    tgts = [tgt_v[pl.ds(q * L, L)] for q in range(NG)]
    colt = [lax.mul(lax.div(t, 128), 128) for t in tgts]
    sub16 = [lax.mul(lax.div(lax.rem(t, 128), L), L) for t in tgts]
    offs = [lax.rem(t, L) for t in tgts]
    rews = [rew_v[pl.ds(q * L, L)] for q in range(NG)]

    def fire(c):
        buf = c % 2
        cps = []
        for j in range(CH):
            e = c * CH + j
            q, i = e // L, e % L
            row8 = pl.multiple_of(base + (e // 8) * 8, 8)
            col = pl.multiple_of(colt[q][i], 128)
            cps.append(pltpu.async_copy(
                prob_hbm.at[pl.ds(row8, 8), pl.ds(col, 128)],
                vals_v.at[buf, j], sems[buf]))
        return cps

    def compute(c, acc):
        buf = c % 2
        for j in range(CH):
            e = c * CH + j
            q, i = e // L, e % L
            row = vals_v[buf, j, e % 8, pl.ds(sub16[q][i], L)]
            sel = jnp.where(lane == offs[q][i], rews[q][i], 0.0)
            acc = acc + row * sel
        return acc

    acc = jnp.zeros((L,), jnp.float32)
    inflight = fire(0)
    for c in range(NCHK):
        nxt = fire(c + 1) if c + 1 < NCHK else None
        for cp in inflight:
            cp.wait()
        acc = compute(c, acc)
        inflight = nxt
    acc_v[...] = acc

    pltpu.sync_copy(acc_v, part_hbm.at[wid])
    plsc.subcore_barrier()

    @pl.when(sid == 0)
    def _():
        pltpu.sync_copy(part_hbm.at[pl.ds(cid * NS, NS)], all_v)
        tot = jnp.zeros((L,), jnp.float32)
        for r in range(NS):
            tot = tot + all_v[r]
        s = tot[0]
        for i in range(1, L):
            s = s + tot[i]
        acc_v[...] = lax.broadcast_in_dim(-s, (L,), ())
        pltpu.sync_copy(acc_v, out_hbm.at[cid])


@jax.jit
def _sc_loss(prob, target, reward):
    mesh = plsc.VectorSubcoreMesh(core_axis_name="c", subcore_axis_name="s")
    f = functools.partial(
        pl.kernel,
        out_type=(jax.ShapeDtypeStruct((NW, L), jnp.float32),
                  jax.ShapeDtypeStruct((NC, L), jnp.float32)),
        mesh=mesh,
        scratch_types=[
            pltpu.VMEM((BW,), jnp.int32),
            pltpu.VMEM((BW,), jnp.float32),
            pltpu.VMEM((2, CH, 8, 128), jnp.float32),
            pltpu.VMEM((L,), jnp.float32),
            pltpu.VMEM((NS, L), jnp.float32),
            pltpu.SemaphoreType.DMA,
            pltpu.SemaphoreType.DMA,
        ],
    )(_sc_body)
    return f(prob, target, reward)


def kernel(prob, target, reward):
    _, out = _sc_loss(prob, target, reward)
    return out[0, 0] + out[1, 0]
